# Initial kernel scaffold; baseline (speedup 1.0000x reference)
import jax
import jax.numpy as jnp
from jax.experimental import pallas as pl


def kernel(ev_features, cs_features, tr_features, env_features, edge_index, ev_indexes, cs_indexes, tr_indexes, env_indexes, sample_node_length, W_ev, b_ev, W_cs, b_cs, W_tr, b_tr, W_env, b_env, W1, b1, W2, b2, W3, b3):
    raise NotImplementedError("write your pallas kernel here")


# speedup vs baseline: 8.0603x; 8.0603x over previous
import functools

import jax
import jax.numpy as jnp
from jax import lax
from jax.experimental import pallas as pl
from jax.experimental.pallas import tpu as pltpu
from jax.experimental.pallas import tpu_sc as plsc

N = 10000
NPAD = 10240
E = 320000
B = 10
NC, NS = 2, 16
NW = NC * NS
C = 128
KCH = 79
EPAD = NW * KCH * C
RPT = NPAD // NS

_f32 = jnp.float32
@functools.cache
def _mesh():
    return plsc.VectorSubcoreMesh(core_axis_name="c", subcore_axis_name="s",
                                  num_cores=NC, num_subcores=NS)


def _x0_body(ev, cs, tr, env, wev, bev, wcs, bcs, wtr, btr, wenv, benv, out):
    out[0:4000, :] = jnp.dot(ev[...], wev[...], preferred_element_type=_f32) + bev[...]
    out[4000:6000, :] = jnp.dot(cs[...], wcs[...], preferred_element_type=_f32) + bcs[...]
    out[6000:8000, :] = jnp.dot(tr[...], wtr[...], preferred_element_type=_f32) + btr[...]
    out[8000:10000, :] = jnp.dot(env[...], wenv[...], preferred_element_type=_f32) + benv[...]
    out[10000:NPAD, :] = jnp.zeros((NPAD - 10000, 128), _f32)


def _build_x0(ev, cs, tr, env, wev, bev, wcs, bcs, wtr, btr, wenv, benv):
    return pl.pallas_call(
        _x0_body,
        out_shape=jax.ShapeDtypeStruct((NPAD, 128), _f32),
    )(ev, cs, tr, env, wev, bev.reshape(1, 128), wcs, bcs.reshape(1, 128),
      wtr, btr.reshape(1, 128), wenv, benv.reshape(1, 128))


def _deg_body(dstp_hbm, out_hbm, idx_v, ones_v, zb_v, acc_sh):
    c = lax.axis_index("c")
    s = lax.axis_index("s")
    w = c * NS + s

    def _fill_z(i, _):
        zb_v[pl.ds(i * 16, 16)] = jnp.zeros((16,), _f32)
        return 0

    lax.fori_loop(0, RPT // 16, _fill_z, 0)

    def _fill_o(i, _):
        ones_v[pl.ds(i * 16, 16)] = jnp.ones((16,), _f32)
        return 0

    lax.fori_loop(0, C // 16, _fill_o, 0)

    pltpu.sync_copy(zb_v, acc_sh.at[pl.ds(s * RPT, RPT)])
    plsc.subcore_barrier()
    pltpu.sync_copy(dstp_hbm.at[w], idx_v)

    def _chunk(k, _):
        pltpu.sync_copy(ones_v, acc_sh.at[idx_v.at[k]], add=True)
        return 0

    lax.fori_loop(0, KCH, _chunk, 0)
    plsc.subcore_barrier()
    pltpu.sync_copy(acc_sh.at[pl.ds(s * RPT, RPT)],
                    out_hbm.at[c].at[pl.ds(s * RPT, RPT)])


@functools.cache
def _deg_kernel():
    return pl.kernel(
        _deg_body,
        out_type=jax.ShapeDtypeStruct((NC, NPAD), _f32),
        mesh=_mesh(),
        scratch_types=[
            pltpu.VMEM((KCH, C), jnp.int32),
            pltpu.VMEM((C,), _f32),
            pltpu.VMEM((RPT,), _f32),
            pltpu.VMEM_SHARED((NPAD,), _f32),
        ],
    )


def _dinv_body(degp, out):
    i = pl.program_id(0)
    rows = i * 128 + lax.broadcasted_iota(jnp.int32, (128, 1), 0)
    d = degp[0] + degp[1] + 1.0
    out[...] = jnp.where(rows < N, lax.rsqrt(d), 0.0)


def _build_dinv(degp):
    return pl.pallas_call(
        _dinv_body,
        grid=(NPAD // 128,),
        in_specs=[pl.BlockSpec((2, 128, 1), lambda i: (0, i, 0))],
        out_specs=pl.BlockSpec((128, 1), lambda i: (i, 0)),
        out_shape=jax.ShapeDtypeStruct((NPAD, 1), _f32),
    )(degp.reshape(NC, NPAD, 1))


def _mm_body(x, w, dinv, h_out, g_out):
    h = jnp.dot(x[...], w[...], preferred_element_type=_f32)
    h_out[0] = h
    g_out[0] = h * dinv[...]


def _build_hg(x, w, dinv, fin, pout):
    h, g = pl.pallas_call(
        _mm_body,
        grid=(NPAD // 128, pout),
        in_specs=[
            pl.BlockSpec((128, fin), lambda i, j: (i, 0)),
            pl.BlockSpec((fin, 128), lambda i, j: (0, j)),
            pl.BlockSpec((128, 1), lambda i, j: (i, 0)),
        ],
        out_specs=[
            pl.BlockSpec((1, 128, 128), lambda i, j: (j, i, 0)),
            pl.BlockSpec((1, 128, 128), lambda i, j: (j, i, 0)),
        ],
        out_shape=[
            jax.ShapeDtypeStruct((pout, NPAD, 128), _f32),
            jax.ShapeDtypeStruct((pout, NPAD, 128), _f32),
        ],
    )(x, w, dinv)
    return h, g


def _agg_body(P, g_hbm, srcp_hbm, dstp_hbm, out_hbm,
              idxs_v, idxd_v, stage_v, zb_v, acc_sh):
    c = lax.axis_index("c")
    s = lax.axis_index("s")
    w = c * NS + s

    def _fill_z(i, _):
        zb_v[i // 8, pl.ds((i % 8) * 16, 16)] = jnp.zeros((16,), _f32)
        return 0

    lax.fori_loop(0, 64 * 8, _fill_z, 0)
    pltpu.sync_copy(srcp_hbm.at[w], idxs_v)
    pltpu.sync_copy(dstp_hbm.at[w], idxd_v)

    for p in range(P):
        def _zslice(r, _):
            pltpu.sync_copy(zb_v, acc_sh.at[pl.ds(s * RPT + r * 64, 64)])
            return 0

        lax.fori_loop(0, RPT // 64, _zslice, 0)
        plsc.subcore_barrier()
        gp = g_hbm.at[p]

        def _chunk(k, _):
            pltpu.sync_copy(gp.at[idxs_v.at[k]], stage_v)
            pltpu.sync_copy(stage_v, acc_sh.at[idxd_v.at[k]], add=True)
            return 0

        lax.fori_loop(0, KCH, _chunk, 0)
        plsc.subcore_barrier()
        pltpu.sync_copy(acc_sh.at[pl.ds(s * RPT, RPT)],
                        out_hbm.at[c].at[p].at[pl.ds(s * RPT, RPT)])
        plsc.subcore_barrier()


@functools.cache
def _make_agg(P):
    return pl.kernel(
        functools.partial(_agg_body, P),
        out_type=jax.ShapeDtypeStruct((NC, P, NPAD, 128), _f32),
        mesh=_mesh(),
        scratch_types=[
            pltpu.VMEM((KCH, C), jnp.int32),
            pltpu.VMEM((KCH, C), jnp.int32),
            pltpu.VMEM((C, 128), _f32),
            pltpu.VMEM((64, 128), _f32),
            pltpu.VMEM_SHARED((NPAD, 128), _f32),
        ],
    )


def _fin_body(part, h, dinv, b, out):
    dv = dinv[...]
    agg = part[0, 0] + part[1, 0]
    o = dv * agg + (dv * dv) * h[0] + b[0]
    out[...] = jnp.maximum(o, 0.0)


def _build_next_x(part, h, dinv, b, pout):
    return pl.pallas_call(
        _fin_body,
        grid=(NPAD // 128, pout),
        in_specs=[
            pl.BlockSpec((2, 1, 128, 128), lambda i, j: (0, j, i, 0)),
            pl.BlockSpec((1, 128, 128), lambda i, j: (j, i, 0)),
            pl.BlockSpec((128, 1), lambda i, j: (i, 0)),
            pl.BlockSpec((1, 1, 128), lambda i, j: (j, 0, 0)),
        ],
        out_specs=pl.BlockSpec((128, 128), lambda i, j: (i, j)),
        out_shape=jax.ShapeDtypeStruct((NPAD, 128 * pout), _f32),
    )(part, h, dinv, b.reshape(pout, 1, 128))


def _pool_body(x, out):
    out[0] = jnp.sum(x[...], axis=0, keepdims=True)


def _build_pool(x):
    out = pl.pallas_call(
        _pool_body,
        grid=(B, 4),
        in_specs=[pl.BlockSpec((N // B, 128), lambda i, j: (i, j))],
        out_specs=pl.BlockSpec((1, 1, 128), lambda i, j: (i, 0, j)),
        out_shape=jax.ShapeDtypeStruct((B, 1, 512), _f32),
    )(x[:N])
    return out.reshape(B, 512)


def kernel(ev_features, cs_features, tr_features, env_features, edge_index,
           ev_indexes, cs_indexes, tr_indexes, env_indexes, sample_node_length,
           W_ev, b_ev, W_cs, b_cs, W_tr, b_tr, W_env, b_env,
           W1, b1, W2, b2, W3, b3):
    src = edge_index[0].astype(jnp.int32)
    dst = edge_index[1].astype(jnp.int32)
    padv = N + (jnp.arange(EPAD - E, dtype=jnp.int32) % (NPAD - N))
    srcp = jnp.concatenate([src, padv]).reshape(NW, KCH, C)
    dstp = jnp.concatenate([dst, padv]).reshape(NW, KCH, C)

    x = _build_x0(ev_features, cs_features, tr_features, env_features,
                  W_ev, b_ev, W_cs, b_cs, W_tr, b_tr, W_env, b_env)
    degp = _deg_kernel()(dstp)
    dinv = _build_dinv(degp)

    for wgt, bias, fin, pout in ((W1, b1, 128, 1), (W2, b2, 128, 2),
                                 (W3, b3, 256, 4)):
        h, g = _build_hg(x, wgt, dinv, fin, pout)
        part = _make_agg(pout)(g, srcp, dstp)
        x = _build_next_x(part, h, dinv, bias, pout)

    sums = _build_pool(x)
    counts = sample_node_length.astype(_f32)
    return sums / counts[:, None]


# speedup vs baseline: 11.4443x; 1.4198x over previous
import functools

import jax
import jax.numpy as jnp
from jax import lax
from jax.experimental import pallas as pl
from jax.experimental.pallas import tpu as pltpu
from jax.experimental.pallas import tpu_sc as plsc

N = 10000
NPAD = 10240
E = 320000
B = 10
NC, NS = 2, 16
NW = NC * NS
C = 128
KCH = 80
KH = KCH // 2
EPAD = NW * KCH * C
RPT = NPAD // NS
NBUF = 2

_f32 = jnp.float32
@functools.cache
def _mesh():
    return plsc.VectorSubcoreMesh(core_axis_name="c", subcore_axis_name="s",
                                  num_cores=NC, num_subcores=NS)


def _x0_body(ev, cs, tr, env, wev, bev, wcs, bcs, wtr, btr, wenv, benv, out):
    out[0:4000, :] = jnp.dot(ev[...], wev[...], preferred_element_type=_f32) + bev[...]
    out[4000:6000, :] = jnp.dot(cs[...], wcs[...], preferred_element_type=_f32) + bcs[...]
    out[6000:8000, :] = jnp.dot(tr[...], wtr[...], preferred_element_type=_f32) + btr[...]
    out[8000:10000, :] = jnp.dot(env[...], wenv[...], preferred_element_type=_f32) + benv[...]
    out[10000:NPAD, :] = jnp.zeros((NPAD - 10000, 128), _f32)


def _build_x0(ev, cs, tr, env, wev, bev, wcs, bcs, wtr, btr, wenv, benv):
    return pl.pallas_call(
        _x0_body,
        out_shape=jax.ShapeDtypeStruct((NPAD, 128), _f32),
    )(ev, cs, tr, env, wev, bev.reshape(1, 128), wcs, bcs.reshape(1, 128),
      wtr, btr.reshape(1, 128), wenv, benv.reshape(1, 128))


def _deg_body(dstp_hbm, out_hbm, idx_v, ones_v, zb_v, acc_sh):
    c = lax.axis_index("c")
    s = lax.axis_index("s")

    def _fill_z(i, _):
        zb_v[pl.ds(i * 16, 16)] = jnp.zeros((16,), _f32)
        return 0

    lax.fori_loop(0, RPT // 16, _fill_z, 0)

    def _fill_o(i, _):
        ones_v[pl.ds(i * 16, 16)] = jnp.ones((16,), _f32)
        return 0

    lax.fori_loop(0, C // 16, _fill_o, 0)

    pltpu.sync_copy(zb_v, acc_sh.at[pl.ds(s * RPT, RPT)])
    plsc.subcore_barrier()
    w = c * NS + s
    pltpu.sync_copy(dstp_hbm.at[w], idx_v)

    def _chunk(k, _):
        pltpu.sync_copy(ones_v, acc_sh.at[idx_v.at[k]], add=True)
        return 0

    lax.fori_loop(0, KCH, _chunk, 0)
    plsc.subcore_barrier()
    pltpu.sync_copy(acc_sh.at[pl.ds(s * RPT, RPT)],
                    out_hbm.at[c].at[pl.ds(s * RPT, RPT)])


@functools.cache
def _deg_kernel():
    return pl.kernel(
        _deg_body,
        out_type=jax.ShapeDtypeStruct((NC, NPAD), _f32),
        mesh=_mesh(),
        scratch_types=[
            pltpu.VMEM((KCH, C), jnp.int32),
            pltpu.VMEM((C,), _f32),
            pltpu.VMEM((RPT,), _f32),
            pltpu.VMEM_SHARED((NPAD,), _f32),
        ],
    )


def _dinv_from(degp, i):
    rows = i * 128 + lax.broadcasted_iota(jnp.int32, (128, 1), 0)
    return jnp.where(rows < N, lax.rsqrt(degp[0] + degp[1] + 1.0), 0.0)


def _mm_body(x, w, degp, h_out, g_out):
    dv = _dinv_from(degp, pl.program_id(0))
    h = jnp.dot(x[...], w[...], preferred_element_type=_f32)
    h_out[0] = h
    g_out[0] = h * dv


def _build_hg(x, w, degp, fin, pout):
    h, g = pl.pallas_call(
        _mm_body,
        grid=(NPAD // 128, pout),
        in_specs=[
            pl.BlockSpec((128, fin), lambda i, j: (i, 0)),
            pl.BlockSpec((fin, 128), lambda i, j: (0, j)),
            pl.BlockSpec((2, 128, 1), lambda i, j: (0, i, 0)),
        ],
        out_specs=[
            pl.BlockSpec((1, 128, 128), lambda i, j: (j, i, 0)),
            pl.BlockSpec((1, 128, 128), lambda i, j: (j, i, 0)),
        ],
        out_shape=[
            jax.ShapeDtypeStruct((pout, NPAD, 128), _f32),
            jax.ShapeDtypeStruct((pout, NPAD, 128), _f32),
        ],
    )(x, w, degp)
    return h, g


def _step_body(pin, part, h_in, degp, b, w, h_out, g_out):
    dv = _dinv_from(degp, pl.program_id(0))
    cols = []
    for p in range(pin):
        agg = part[0, p] + part[1, p]
        cols.append(jnp.maximum(dv * agg + (dv * dv) * h_in[p] + b[p], 0.0))
    xb = jnp.concatenate(cols, axis=1) if pin > 1 else cols[0]
    h = jnp.dot(xb, w[...], preferred_element_type=_f32)
    h_out[0] = h
    g_out[0] = h * dv


def _build_step(part, h_in, degp, b, w, pin, pout):
    h, g = pl.pallas_call(
        functools.partial(_step_body, pin),
        grid=(NPAD // 128, pout),
        in_specs=[
            pl.BlockSpec((2, pin, 128, 128), lambda i, j: (0, 0, i, 0)),
            pl.BlockSpec((pin, 128, 128), lambda i, j: (0, i, 0)),
            pl.BlockSpec((2, 128, 1), lambda i, j: (0, i, 0)),
            pl.BlockSpec((pin, 1, 128), lambda i, j: (0, 0, 0)),
            pl.BlockSpec((128 * pin, 128), lambda i, j: (0, j)),
        ],
        out_specs=[
            pl.BlockSpec((1, 128, 128), lambda i, j: (j, i, 0)),
            pl.BlockSpec((1, 128, 128), lambda i, j: (j, i, 0)),
        ],
        out_shape=[
            jax.ShapeDtypeStruct((pout, NPAD, 128), _f32),
            jax.ShapeDtypeStruct((pout, NPAD, 128), _f32),
        ],
    )(part, h_in, degp, b.reshape(pin, 1, 128), w)
    return h, g


def _agg_body(P, g_hbm, srcp_hbm, dstp_hbm, out_hbm,
              idxs_v, idxd_v, stage_v, zb_v, acc_sh, sem0, sem1):
    c = lax.axis_index("c")
    s = lax.axis_index("s")
    w = c * NS + s
    sems = (sem0, sem1)

    def _fill_z(i, _):
        zb_v[i // 8, pl.ds((i % 8) * 16, 16)] = jnp.zeros((16,), _f32)
        return 0

    lax.fori_loop(0, 32 * 8, _fill_z, 0)

    for p in range(P):
        def _ziss(r, _):
            pltpu.async_copy(zb_v, acc_sh.at[pl.ds(s * RPT + r * 32, 32)],
                             sem0)
            return 0

        def _zwait(r, _):
            pltpu.make_async_copy(zb_v, acc_sh.at[pl.ds(s * RPT, 32)],
                                  sem0).wait()
            return 0

        lax.fori_loop(0, RPT // 32, _ziss, 0)
        lax.fori_loop(0, RPT // 32, _zwait, 0)
        plsc.subcore_barrier()
        gp = g_hbm.at[p]

        for hf in range(2):
            pltpu.sync_copy(srcp_hbm.at[w].at[pl.ds(hf * KH, KH)], idxs_v)
            pltpu.sync_copy(dstp_hbm.at[w].at[pl.ds(hf * KH, KH)], idxd_v)

            for b in range(NBUF):
                pltpu.async_copy(gp.at[idxs_v.at[b]], stage_v.at[b], sems[b])

            def _ring(k0, _):
                for b in range(NBUF):
                    k = k0 * NBUF + b
                    pltpu.make_async_copy(gp.at[idxs_v.at[k]], stage_v.at[b],
                                          sems[b]).wait()
                    pltpu.sync_copy(stage_v.at[b], acc_sh.at[idxd_v.at[k]],
                                    add=True)
                    pltpu.async_copy(gp.at[idxs_v.at[k + NBUF]],
                                     stage_v.at[b], sems[b])
                return 0

            lax.fori_loop(0, KH // NBUF - 1, _ring, 0)
            for b in range(NBUF):
                k = KH - NBUF + b
                pltpu.make_async_copy(gp.at[idxs_v.at[k]], stage_v.at[b],
                                      sems[b]).wait()
                pltpu.sync_copy(stage_v.at[b], acc_sh.at[idxd_v.at[k]],
                                add=True)

        plsc.subcore_barrier()
        pltpu.sync_copy(acc_sh.at[pl.ds(s * RPT, RPT)],
                        out_hbm.at[c].at[p].at[pl.ds(s * RPT, RPT)])
        plsc.subcore_barrier()


@functools.cache
def _make_agg(P):
    return pl.kernel(
        functools.partial(_agg_body, P),
        out_type=jax.ShapeDtypeStruct((NC, P, NPAD, 128), _f32),
        mesh=_mesh(),
        scratch_types=[
            pltpu.VMEM((KH, C), jnp.int32),
            pltpu.VMEM((KH, C), jnp.int32),
            pltpu.VMEM((NBUF, C, 128), _f32),
            pltpu.VMEM((32, 128), _f32),
            pltpu.VMEM_SHARED((NPAD, 128), _f32),
            pltpu.SemaphoreType.DMA,
            pltpu.SemaphoreType.DMA,
        ],
    )


def _fin_body(part, h, degp, b, out):
    dv = _dinv_from(degp, pl.program_id(0))
    agg = part[0, 0] + part[1, 0]
    o = dv * agg + (dv * dv) * h[0] + b[0]
    out[...] = jnp.maximum(o, 0.0)


def _build_next_x(part, h, degp, b, pout):
    return pl.pallas_call(
        _fin_body,
        grid=(NPAD // 128, pout),
        in_specs=[
            pl.BlockSpec((2, 1, 128, 128), lambda i, j: (0, j, i, 0)),
            pl.BlockSpec((1, 128, 128), lambda i, j: (j, i, 0)),
            pl.BlockSpec((2, 128, 1), lambda i, j: (0, i, 0)),
            pl.BlockSpec((1, 1, 128), lambda i, j: (j, 0, 0)),
        ],
        out_specs=pl.BlockSpec((128, 128), lambda i, j: (i, j)),
        out_shape=jax.ShapeDtypeStruct((NPAD, 128 * pout), _f32),
    )(part, h, degp, b.reshape(pout, 1, 128))


def _pool_body(x, out):
    out[0] = jnp.sum(x[...], axis=0, keepdims=True)


def _build_pool(x):
    out = pl.pallas_call(
        _pool_body,
        grid=(B, 4),
        in_specs=[pl.BlockSpec((N // B, 128), lambda i, j: (i, j))],
        out_specs=pl.BlockSpec((1, 1, 128), lambda i, j: (i, 0, j)),
        out_shape=jax.ShapeDtypeStruct((B, 1, 512), _f32),
    )(x[:N])
    return out.reshape(B, 512)


def kernel(ev_features, cs_features, tr_features, env_features, edge_index,
           ev_indexes, cs_indexes, tr_indexes, env_indexes, sample_node_length,
           W_ev, b_ev, W_cs, b_cs, W_tr, b_tr, W_env, b_env,
           W1, b1, W2, b2, W3, b3):
    src = edge_index[0].astype(jnp.int32)
    dst = edge_index[1].astype(jnp.int32)
    padv = N + (jnp.arange(EPAD - E, dtype=jnp.int32) % (NPAD - N))
    srcp = jnp.concatenate([src, padv]).reshape(NW, KCH, C)
    dstp = jnp.concatenate([dst, padv]).reshape(NW, KCH, C)

    x0 = _build_x0(ev_features, cs_features, tr_features, env_features,
                   W_ev, b_ev, W_cs, b_cs, W_tr, b_tr, W_env, b_env)
    degp = _deg_kernel()(dstp).reshape(NC, NPAD, 1)

    h, g = _build_hg(x0, W1, degp, 128, 1)
    part = _make_agg(1)(g, srcp, dstp)
    h, g = _build_step(part, h, degp, b1, W2, pin=1, pout=2)
    part = _make_agg(2)(g, srcp, dstp)
    h, g = _build_step(part, h, degp, b2, W3, pin=2, pout=4)
    part = _make_agg(4)(g, srcp, dstp)
    x3 = _build_next_x(part, h, degp, b3, pout=4)

    sums = _build_pool(x3)
    counts = sample_node_length.astype(_f32)
    return sums / counts[:, None]


# speedup vs baseline: 16.2650x; 1.4212x over previous
import functools

import jax
import jax.numpy as jnp
from jax import lax
from jax.experimental import pallas as pl
from jax.experimental.pallas import tpu as pltpu
from jax.experimental.pallas import tpu_sc as plsc

N = 10000
NPAD = 10240
E = 320000
B = 10
NC, NS = 2, 16
NW = NC * NS
C = 128
KCH = 80
KH = KCH // 2
EPAD = NW * KCH * C
RPT = NPAD // NS
NBUF = 2

_f32 = jnp.float32
@functools.cache
def _mesh():
    return plsc.VectorSubcoreMesh(core_axis_name="c", subcore_axis_name="s",
                                  num_cores=NC, num_subcores=NS)


def _x0_body(ev, cs, tr, env, wev, bev, wcs, bcs, wtr, btr, wenv, benv, out):
    out[0:4000, :] = jnp.dot(ev[...], wev[...], preferred_element_type=_f32) + bev[...]
    out[4000:6000, :] = jnp.dot(cs[...], wcs[...], preferred_element_type=_f32) + bcs[...]
    out[6000:8000, :] = jnp.dot(tr[...], wtr[...], preferred_element_type=_f32) + btr[...]
    out[8000:10000, :] = jnp.dot(env[...], wenv[...], preferred_element_type=_f32) + benv[...]
    out[10000:NPAD, :] = jnp.zeros((NPAD - 10000, 128), _f32)


def _build_x0(ev, cs, tr, env, wev, bev, wcs, bcs, wtr, btr, wenv, benv):
    return pl.pallas_call(
        _x0_body,
        out_shape=jax.ShapeDtypeStruct((NPAD, 128), _f32),
    )(ev, cs, tr, env, wev, bev.reshape(1, 128), wcs, bcs.reshape(1, 128),
      wtr, btr.reshape(1, 128), wenv, benv.reshape(1, 128))


def _deg_body(dstp_hbm, out_hbm, idx_v, ones_v, zb_v, acc_sh):
    c = lax.axis_index("c")
    s = lax.axis_index("s")

    def _fill_z(i, _):
        zb_v[pl.ds(i * 16, 16)] = jnp.zeros((16,), _f32)
        return 0

    lax.fori_loop(0, RPT // 16, _fill_z, 0)

    def _fill_o(i, _):
        ones_v[pl.ds(i * 16, 16)] = jnp.ones((16,), _f32)
        return 0

    lax.fori_loop(0, C // 16, _fill_o, 0)

    pltpu.sync_copy(zb_v, acc_sh.at[pl.ds(s * RPT, RPT)])
    plsc.subcore_barrier()
    w = c * NS + s
    pltpu.sync_copy(dstp_hbm.at[w], idx_v)

    def _chunk(k, _):
        pltpu.sync_copy(ones_v, acc_sh.at[idx_v.at[k]], add=True)
        return 0

    lax.fori_loop(0, KCH, _chunk, 0)
    plsc.subcore_barrier()
    pltpu.sync_copy(acc_sh.at[pl.ds(s * RPT, RPT)],
                    out_hbm.at[c].at[pl.ds(s * RPT, RPT)])


@functools.cache
def _deg_kernel():
    return pl.kernel(
        _deg_body,
        out_type=jax.ShapeDtypeStruct((NC, NPAD), _f32),
        mesh=_mesh(),
        scratch_types=[
            pltpu.VMEM((KCH, C), jnp.int32),
            pltpu.VMEM((C,), _f32),
            pltpu.VMEM((RPT,), _f32),
            pltpu.VMEM_SHARED((NPAD,), _f32),
        ],
    )


RB = 1024


def _dinv_from(degp, i):
    rows = i * RB + lax.broadcasted_iota(jnp.int32, (RB, 1), 0)
    return jnp.where(rows < N, lax.rsqrt(degp[0] + degp[1] + 1.0), 0.0)


def _mm_body(x, w, degp, h_out, g_out):
    dv = _dinv_from(degp, pl.program_id(0))
    h = jnp.dot(x[...], w[...], preferred_element_type=_f32)
    h_out[0] = h
    g_out[0] = h * dv


def _build_hg(x, w, degp, fin, pout):
    h, g = pl.pallas_call(
        _mm_body,
        grid=(NPAD // RB, pout),
        in_specs=[
            pl.BlockSpec((RB, fin), lambda i, j: (i, 0)),
            pl.BlockSpec((fin, 128), lambda i, j: (0, j)),
            pl.BlockSpec((2, RB, 1), lambda i, j: (0, i, 0)),
        ],
        out_specs=[
            pl.BlockSpec((1, RB, 128), lambda i, j: (j, i, 0)),
            pl.BlockSpec((1, RB, 128), lambda i, j: (j, i, 0)),
        ],
        out_shape=[
            jax.ShapeDtypeStruct((pout, NPAD, 128), _f32),
            jax.ShapeDtypeStruct((pout, NPAD, 128), _f32),
        ],
    )(x, w, degp)
    return h, g


def _step_body(pin, part, h_in, degp, b, w, h_out, g_out):
    dv = _dinv_from(degp, pl.program_id(0))
    cols = []
    for p in range(pin):
        agg = part[0, p] + part[1, p]
        cols.append(jnp.maximum(dv * agg + (dv * dv) * h_in[p] + b[p], 0.0))
    xb = jnp.concatenate(cols, axis=1) if pin > 1 else cols[0]
    h = jnp.dot(xb, w[...], preferred_element_type=_f32)
    h_out[0] = h
    g_out[0] = h * dv


def _build_step(part, h_in, degp, b, w, pin, pout):
    h, g = pl.pallas_call(
        functools.partial(_step_body, pin),
        grid=(NPAD // RB, pout),
        in_specs=[
            pl.BlockSpec((2, pin, RB, 128), lambda i, j: (0, 0, i, 0)),
            pl.BlockSpec((pin, RB, 128), lambda i, j: (0, i, 0)),
            pl.BlockSpec((2, RB, 1), lambda i, j: (0, i, 0)),
            pl.BlockSpec((pin, 1, 128), lambda i, j: (0, 0, 0)),
            pl.BlockSpec((128 * pin, 128), lambda i, j: (0, j)),
        ],
        out_specs=[
            pl.BlockSpec((1, RB, 128), lambda i, j: (j, i, 0)),
            pl.BlockSpec((1, RB, 128), lambda i, j: (j, i, 0)),
        ],
        out_shape=[
            jax.ShapeDtypeStruct((pout, NPAD, 128), _f32),
            jax.ShapeDtypeStruct((pout, NPAD, 128), _f32),
        ],
    )(part, h_in, degp, b.reshape(pin, 1, 128), w)
    return h, g


def _agg_body(P, g_hbm, srcp_hbm, dstp_hbm, out_hbm,
              idxs_v, idxd_v, stage_v, zb_v, acc_sh, sem0, sem1):
    c = lax.axis_index("c")
    s = lax.axis_index("s")
    w = c * NS + s
    sems = (sem0, sem1)

    def _fill_z(i, _):
        zb_v[i // 8, pl.ds((i % 8) * 16, 16)] = jnp.zeros((16,), _f32)
        return 0

    lax.fori_loop(0, 32 * 8, _fill_z, 0)

    for p in range(P):
        def _ziss(r, _):
            pltpu.async_copy(zb_v, acc_sh.at[pl.ds(s * RPT + r * 32, 32)],
                             sem0)
            return 0

        def _zwait(r, _):
            pltpu.make_async_copy(zb_v, acc_sh.at[pl.ds(s * RPT, 32)],
                                  sem0).wait()
            return 0

        lax.fori_loop(0, RPT // 32, _ziss, 0)
        lax.fori_loop(0, RPT // 32, _zwait, 0)
        plsc.subcore_barrier()
        gp = g_hbm.at[p]

        for hf in range(2):
            pltpu.sync_copy(srcp_hbm.at[w].at[pl.ds(hf * KH, KH)], idxs_v)
            pltpu.sync_copy(dstp_hbm.at[w].at[pl.ds(hf * KH, KH)], idxd_v)

            for b in range(NBUF):
                pltpu.async_copy(gp.at[idxs_v.at[b]], stage_v.at[b], sems[b])

            def _ring(k0, _):
                for b in range(NBUF):
                    k = k0 * NBUF + b
                    pltpu.make_async_copy(gp.at[idxs_v.at[k]], stage_v.at[b],
                                          sems[b]).wait()
                    pltpu.sync_copy(stage_v.at[b], acc_sh.at[idxd_v.at[k]],
                                    add=True)
                    pltpu.async_copy(gp.at[idxs_v.at[k + NBUF]],
                                     stage_v.at[b], sems[b])
                return 0

            lax.fori_loop(0, KH // NBUF - 1, _ring, 0)
            for b in range(NBUF):
                k = KH - NBUF + b
                pltpu.make_async_copy(gp.at[idxs_v.at[k]], stage_v.at[b],
                                      sems[b]).wait()
                pltpu.sync_copy(stage_v.at[b], acc_sh.at[idxd_v.at[k]],
                                add=True)

        plsc.subcore_barrier()
        pltpu.sync_copy(acc_sh.at[pl.ds(s * RPT, RPT)],
                        out_hbm.at[c].at[p].at[pl.ds(s * RPT, RPT)])
        plsc.subcore_barrier()


@functools.cache
def _make_agg(P):
    return pl.kernel(
        functools.partial(_agg_body, P),
        out_type=jax.ShapeDtypeStruct((NC, P, NPAD, 128), _f32),
        mesh=_mesh(),
        scratch_types=[
            pltpu.VMEM((KH, C), jnp.int32),
            pltpu.VMEM((KH, C), jnp.int32),
            pltpu.VMEM((NBUF, C, 128), _f32),
            pltpu.VMEM((32, 128), _f32),
            pltpu.VMEM_SHARED((NPAD, 128), _f32),
            pltpu.SemaphoreType.DMA,
            pltpu.SemaphoreType.DMA,
        ],
    )


def _fin_body(part, h, degp, b, out):
    dv = _dinv_from(degp, pl.program_id(0))
    agg = part[0, 0] + part[1, 0]
    o = dv * agg + (dv * dv) * h[0] + b[0]
    out[...] = jnp.maximum(o, 0.0)


def _build_next_x(part, h, degp, b, pout):
    return pl.pallas_call(
        _fin_body,
        grid=(NPAD // RB, pout),
        in_specs=[
            pl.BlockSpec((2, 1, RB, 128), lambda i, j: (0, j, i, 0)),
            pl.BlockSpec((1, RB, 128), lambda i, j: (j, i, 0)),
            pl.BlockSpec((2, RB, 1), lambda i, j: (0, i, 0)),
            pl.BlockSpec((1, 1, 128), lambda i, j: (j, 0, 0)),
        ],
        out_specs=pl.BlockSpec((RB, 128), lambda i, j: (i, j)),
        out_shape=jax.ShapeDtypeStruct((NPAD, 128 * pout), _f32),
    )(part, h, degp, b.reshape(pout, 1, 128))


def _pool_body(x, out):
    out[0] = jnp.sum(x[...], axis=0, keepdims=True)


def _build_pool(x):
    out = pl.pallas_call(
        _pool_body,
        grid=(B, 4),
        in_specs=[pl.BlockSpec((N // B, 128), lambda i, j: (i, j))],
        out_specs=pl.BlockSpec((1, 1, 128), lambda i, j: (i, 0, j)),
        out_shape=jax.ShapeDtypeStruct((B, 1, 512), _f32),
    )(x[:N])
    return out.reshape(B, 512)


def kernel(ev_features, cs_features, tr_features, env_features, edge_index,
           ev_indexes, cs_indexes, tr_indexes, env_indexes, sample_node_length,
           W_ev, b_ev, W_cs, b_cs, W_tr, b_tr, W_env, b_env,
           W1, b1, W2, b2, W3, b3):
    src = edge_index[0].astype(jnp.int32)
    dst = edge_index[1].astype(jnp.int32)
    padv = N + (jnp.arange(EPAD - E, dtype=jnp.int32) % (NPAD - N))
    srcp = jnp.concatenate([src, padv]).reshape(NW, KCH, C)
    dstp = jnp.concatenate([dst, padv]).reshape(NW, KCH, C)

    x0 = _build_x0(ev_features, cs_features, tr_features, env_features,
                   W_ev, b_ev, W_cs, b_cs, W_tr, b_tr, W_env, b_env)
    degp = _deg_kernel()(dstp).reshape(NC, NPAD, 1)

    h, g = _build_hg(x0, W1, degp, 128, 1)
    part = _make_agg(1)(g, srcp, dstp)
    h, g = _build_step(part, h, degp, b1, W2, pin=1, pout=2)
    part = _make_agg(2)(g, srcp, dstp)
    h, g = _build_step(part, h, degp, b2, W3, pin=2, pout=4)
    part = _make_agg(4)(g, srcp, dstp)
    x3 = _build_next_x(part, h, degp, b3, pout=4)

    sums = _build_pool(x3)
    counts = sample_node_length.astype(_f32)
    return sums / counts[:, None]


# speedup vs baseline: 17.3276x; 1.0653x over previous
import functools

import jax
import jax.numpy as jnp
from jax import lax
from jax.experimental import pallas as pl
from jax.experimental.pallas import tpu as pltpu
from jax.experimental.pallas import tpu_sc as plsc

N = 10000
NPAD = 10240
E = 320000
B = 10
NC, NS = 2, 16
NW = NC * NS
C = 128
KCH = 80
KH = KCH // 2
EPAD = NW * KCH * C
RPT = NPAD // NS
NBUF = 2

_f32 = jnp.float32
@functools.cache
def _mesh():
    return plsc.VectorSubcoreMesh(core_axis_name="c", subcore_axis_name="s",
                                  num_cores=NC, num_subcores=NS)


def _x0_body(ev, cs, tr, env, wev, bev, wcs, bcs, wtr, btr, wenv, benv, out):
    out[0:4000, :] = jnp.dot(ev[...], wev[...], preferred_element_type=_f32) + bev[...]
    out[4000:6000, :] = jnp.dot(cs[...], wcs[...], preferred_element_type=_f32) + bcs[...]
    out[6000:8000, :] = jnp.dot(tr[...], wtr[...], preferred_element_type=_f32) + btr[...]
    out[8000:10000, :] = jnp.dot(env[...], wenv[...], preferred_element_type=_f32) + benv[...]
    out[10000:NPAD, :] = jnp.zeros((NPAD - 10000, 128), _f32)


def _build_x0(ev, cs, tr, env, wev, bev, wcs, bcs, wtr, btr, wenv, benv):
    return pl.pallas_call(
        _x0_body,
        out_shape=jax.ShapeDtypeStruct((NPAD, 128), _f32),
    )(ev, cs, tr, env, wev, bev.reshape(1, 128), wcs, bcs.reshape(1, 128),
      wtr, btr.reshape(1, 128), wenv, benv.reshape(1, 128))


def _deg_body(dstp_hbm, out_hbm, idx_v, ones_v, zb_v, acc_sh):
    c = lax.axis_index("c")
    s = lax.axis_index("s")

    def _fill_z(i, _):
        zb_v[pl.ds(i * 16, 16)] = jnp.zeros((16,), _f32)
        return 0

    lax.fori_loop(0, RPT // 16, _fill_z, 0)

    def _fill_o(i, _):
        ones_v[pl.ds(i * 16, 16)] = jnp.ones((16,), _f32)
        return 0

    lax.fori_loop(0, C // 16, _fill_o, 0)

    pltpu.sync_copy(zb_v, acc_sh.at[pl.ds(s * RPT, RPT)])
    plsc.subcore_barrier()
    w = c * NS + s
    pltpu.sync_copy(dstp_hbm.at[w], idx_v)

    def _chunk(k, _):
        pltpu.sync_copy(ones_v, acc_sh.at[idx_v.at[k]], add=True)
        return 0

    lax.fori_loop(0, KCH, _chunk, 0)
    plsc.subcore_barrier()
    pltpu.sync_copy(acc_sh.at[pl.ds(s * RPT, RPT)],
                    out_hbm.at[c].at[pl.ds(s * RPT, RPT)])


@functools.cache
def _deg_kernel():
    return pl.kernel(
        _deg_body,
        out_type=jax.ShapeDtypeStruct((NC, NPAD), _f32),
        mesh=_mesh(),
        scratch_types=[
            pltpu.VMEM((KCH, C), jnp.int32),
            pltpu.VMEM((C,), _f32),
            pltpu.VMEM((RPT,), _f32),
            pltpu.VMEM_SHARED((NPAD,), _f32),
        ],
    )


RB = 1024


def _dinv_from(degp, i, rb=RB):
    rows = i * rb + lax.broadcasted_iota(jnp.int32, (rb, 1), 0)
    return jnp.where(rows < N, lax.rsqrt(degp[0] + degp[1] + 1.0), 0.0)


def _mm_body(x, w, degp, g_out):
    dv = _dinv_from(degp, pl.program_id(0))
    h = jnp.dot(x[...], w[...], preferred_element_type=_f32)
    g_out[0] = h * dv


def _build_g1(x, w, degp, fin, pout):
    return pl.pallas_call(
        _mm_body,
        grid=(NPAD // RB, pout),
        in_specs=[
            pl.BlockSpec((RB, fin), lambda i, j: (i, 0)),
            pl.BlockSpec((fin, 128), lambda i, j: (0, j)),
            pl.BlockSpec((2, RB, 1), lambda i, j: (0, i, 0)),
        ],
        out_specs=pl.BlockSpec((1, RB, 128), lambda i, j: (j, i, 0)),
        out_shape=jax.ShapeDtypeStruct((pout, NPAD, 128), _f32),
    )(x, w, degp)


def _step_body(pin, part, g_in, degp, b, w, g_out):
    dv = _dinv_from(degp, pl.program_id(0))
    cols = []
    for p in range(pin):
        agg = part[0, p] + part[1, p]
        cols.append(jnp.maximum(dv * (agg + g_in[p]) + b[p], 0.0))
    xb = jnp.concatenate(cols, axis=1) if pin > 1 else cols[0]
    h = jnp.dot(xb, w[...], preferred_element_type=_f32)
    g_out[0] = h * dv


def _build_step(part, g_in, degp, b, w, pin, pout):
    return pl.pallas_call(
        functools.partial(_step_body, pin),
        grid=(NPAD // RB, pout),
        in_specs=[
            pl.BlockSpec((2, pin, RB, 128), lambda i, j: (0, 0, i, 0)),
            pl.BlockSpec((pin, RB, 128), lambda i, j: (0, i, 0)),
            pl.BlockSpec((2, RB, 1), lambda i, j: (0, i, 0)),
            pl.BlockSpec((pin, 1, 128), lambda i, j: (0, 0, 0)),
            pl.BlockSpec((128 * pin, 128), lambda i, j: (0, j)),
        ],
        out_specs=pl.BlockSpec((1, RB, 128), lambda i, j: (j, i, 0)),
        out_shape=jax.ShapeDtypeStruct((pout, NPAD, 128), _f32),
    )(part, g_in, degp, b.reshape(pin, 1, 128), w)


def _agg_body(P, g_hbm, srcp_hbm, dstp_hbm, out_hbm,
              idxs_v, idxd_v, stage_v, zb_v, acc_sh, sem0, sem1):
    c = lax.axis_index("c")
    s = lax.axis_index("s")
    w = c * NS + s
    sems = (sem0, sem1)

    def _fill_z(i, _):
        zb_v[i // 8, pl.ds((i % 8) * 16, 16)] = jnp.zeros((16,), _f32)
        return 0

    lax.fori_loop(0, 32 * 8, _fill_z, 0)

    for p in range(P):
        def _ziss(r, _):
            pltpu.async_copy(zb_v, acc_sh.at[pl.ds(s * RPT + r * 32, 32)],
                             sem0)
            return 0

        def _zwait(r, _):
            pltpu.make_async_copy(zb_v, acc_sh.at[pl.ds(s * RPT, 32)],
                                  sem0).wait()
            return 0

        lax.fori_loop(0, RPT // 32, _ziss, 0)
        lax.fori_loop(0, RPT // 32, _zwait, 0)
        plsc.subcore_barrier()
        gp = g_hbm.at[p]

        for hf in range(2):
            pltpu.sync_copy(srcp_hbm.at[w].at[pl.ds(hf * KH, KH)], idxs_v)
            pltpu.sync_copy(dstp_hbm.at[w].at[pl.ds(hf * KH, KH)], idxd_v)

            for b in range(NBUF):
                pltpu.async_copy(gp.at[idxs_v.at[b]], stage_v.at[b], sems[b])

            def _ring(k0, _):
                for b in range(NBUF):
                    k = k0 * NBUF + b
                    pltpu.make_async_copy(gp.at[idxs_v.at[k]], stage_v.at[b],
                                          sems[b]).wait()
                    pltpu.sync_copy(stage_v.at[b], acc_sh.at[idxd_v.at[k]],
                                    add=True)
                    pltpu.async_copy(gp.at[idxs_v.at[k + NBUF]],
                                     stage_v.at[b], sems[b])
                return 0

            lax.fori_loop(0, KH // NBUF - 1, _ring, 0)
            for b in range(NBUF):
                k = KH - NBUF + b
                pltpu.make_async_copy(gp.at[idxs_v.at[k]], stage_v.at[b],
                                      sems[b]).wait()
                pltpu.sync_copy(stage_v.at[b], acc_sh.at[idxd_v.at[k]],
                                add=True)

        plsc.subcore_barrier()
        pltpu.sync_copy(acc_sh.at[pl.ds(s * RPT, RPT)],
                        out_hbm.at[c].at[p].at[pl.ds(s * RPT, RPT)])
        plsc.subcore_barrier()


@functools.cache
def _make_agg(P):
    return pl.kernel(
        functools.partial(_agg_body, P),
        out_type=jax.ShapeDtypeStruct((NC, P, NPAD, 128), _f32),
        mesh=_mesh(),
        scratch_types=[
            pltpu.VMEM((KH, C), jnp.int32),
            pltpu.VMEM((KH, C), jnp.int32),
            pltpu.VMEM((NBUF, C, 128), _f32),
            pltpu.VMEM((32, 128), _f32),
            pltpu.VMEM_SHARED((NPAD, 128), _f32),
            pltpu.SemaphoreType.DMA,
            pltpu.SemaphoreType.DMA,
        ],
    )


SEG = N // B


def _finpool_body(part, g_in, degp, b, out):
    dv = _dinv_from(degp, pl.program_id(0), SEG)
    agg = part[0, 0] + part[1, 0]
    o = jnp.maximum(dv * (agg + g_in[0]) + b[0], 0.0)
    out[0, 0] = jnp.sum(o, axis=0)


def _build_finpool(part, g_in, degp, b):
    return pl.pallas_call(
        _finpool_body,
        grid=(B, 4),
        in_specs=[
            pl.BlockSpec((2, 1, SEG, 128), lambda i, j: (0, j, i, 0)),
            pl.BlockSpec((1, SEG, 128), lambda i, j: (j, i, 0)),
            pl.BlockSpec((2, SEG, 1), lambda i, j: (0, i, 0)),
            pl.BlockSpec((1, 1, 128), lambda i, j: (j, 0, 0)),
        ],
        out_specs=pl.BlockSpec((1, 1, 128), lambda i, j: (i, 0, j)),
        out_shape=jax.ShapeDtypeStruct((B, 1, 512), _f32),
    )(part, g_in, degp, b.reshape(4, 1, 128)).reshape(B, 512)


def kernel(ev_features, cs_features, tr_features, env_features, edge_index,
           ev_indexes, cs_indexes, tr_indexes, env_indexes, sample_node_length,
           W_ev, b_ev, W_cs, b_cs, W_tr, b_tr, W_env, b_env,
           W1, b1, W2, b2, W3, b3):
    src = edge_index[0].astype(jnp.int32)
    dst = edge_index[1].astype(jnp.int32)
    padv = N + (jnp.arange(EPAD - E, dtype=jnp.int32) % (NPAD - N))
    srcp = jnp.concatenate([src, padv]).reshape(NW, KCH, C)
    dstp = jnp.concatenate([dst, padv]).reshape(NW, KCH, C)

    x0 = _build_x0(ev_features, cs_features, tr_features, env_features,
                   W_ev, b_ev, W_cs, b_cs, W_tr, b_tr, W_env, b_env)
    degp = _deg_kernel()(dstp).reshape(NC, NPAD, 1)

    g = _build_g1(x0, W1, degp, 128, 1)
    part = _make_agg(1)(g, srcp, dstp)
    g = _build_step(part, g, degp, b1, W2, pin=1, pout=2)
    part = _make_agg(2)(g, srcp, dstp)
    g = _build_step(part, g, degp, b2, W3, pin=2, pout=4)
    part = _make_agg(4)(g, srcp, dstp)
    sums = _build_finpool(part, g, degp, b3)

    counts = sample_node_length.astype(_f32)
    return sums / counts[:, None]
